# baseline (speedup 1.0000x reference)
import jax
import jax.numpy as jnp
from jax.experimental import pallas as pl
from jax.experimental.pallas import tpu as pltpu

_C = 0.7071067811865476


def _backbone_kernel(xr_ref, wall_ref, wpw_ref, dsg_ref, dsb_ref,
                     w12_ref, b1_ref, w2_ref, b2_ref, upg_ref, upb_ref,
                     ds_ref, up_ref):
    D = dsg_ref.shape[-1]
    nb = xr_ref.shape[0]
    Tp = xr_ref.shape[1]
    z1 = jnp.zeros((1, D), jnp.float32)
    b1 = b1_ref[...]
    b2 = b2_ref[...]
    gam = upg_ref[...]
    bet = upb_ref[...]
    for i in range(nb):
        X = xr_ref[i]
        A = jnp.dot(X, wall_ref[...], preferred_element_type=jnp.float32)
        acc = (A[:, :D]
               + jnp.concatenate([z1, A[:-1, D:2 * D]], axis=0)
               + jnp.concatenate([A[1:, 2 * D:], z1], axis=0))
        dw = jnp.maximum(acc, 0.0)
        pw = jnp.dot(dw, wpw_ref[...], preferred_element_type=jnp.float32)
        pw = jnp.maximum(pw, 0.0)
        mu = jnp.mean(pw, axis=-1, keepdims=True)
        var = jnp.mean(jnp.square(pw - mu), axis=-1, keepdims=True)
        ds = ((pw - mu) * jax.lax.rsqrt(var + 1e-5) * dsg_ref[...]
              + dsb_ref[...])
        ds_ref[i] = ds

        xn = jnp.concatenate([ds[1:], z1], axis=0)
        X2 = jnp.concatenate([ds, xn], axis=1)
        A2 = jnp.dot(X2, w12_ref[...], preferred_element_type=jnp.float32)
        ge = A2[:, :D] + b1
        go = A2[:, D:] + b1
        h_even = ge * (1.0 + jax.lax.erf(ge))
        h_odd = go * (1.0 + jax.lax.erf(go))

        for j, h in enumerate((h_even, h_odd)):
            Y = jnp.dot(h, w2_ref[...], preferred_element_type=jnp.float32)
            for k in range(5):
                y = Y[:, k * D:(k + 1) * D] + b2
                zz = y * (1.0 + jax.lax.erf(y))
                m2 = jnp.mean(zz, axis=-1, keepdims=True)
                v2 = jnp.mean(jnp.square(zz - m2), axis=-1, keepdims=True)
                z = (zz - m2) * jax.lax.rsqrt(v2 + 2e-5) * gam + bet
                out = (_C * z) * (1.0 + jax.lax.erf(z))
                up_ref[i, (5 * j + k)::10, :] = out


def kernel(x, x_len, dw_w, pw_w, ds_ln_g, ds_ln_b, up_w1, up_b1, up_w2, up_b2,
           up_ln_g, up_ln_b):
    del x_len
    B, T, Cin = x.shape
    D = pw_w.shape[1]
    stride = 10
    Tp = T // stride

    f32 = jnp.float32
    xr = x.astype(jnp.bfloat16).reshape(B, Tp, stride * Cin)

    E = jnp.eye(Cin, dtype=f32)
    blk_c = dw_w[3:13, 0, :, None] * E[None]
    blk_p = jnp.zeros((stride, Cin, Cin), f32).at[7:10].set(
        dw_w[0:3, 0, :, None] * E[None])
    blk_n = jnp.zeros((stride, Cin, Cin), f32).at[0:2].set(
        dw_w[13:15, 0, :, None] * E[None])
    wall = jnp.zeros((stride * Cin, 3 * D), f32)
    wall = wall.at[:, 0:Cin].set(blk_c.reshape(stride * Cin, Cin))
    wall = wall.at[:, D:D + Cin].set(blk_p.reshape(stride * Cin, Cin))
    wall = wall.at[:, 2 * D:2 * D + Cin].set(blk_n.reshape(stride * Cin, Cin))
    wall = wall.astype(jnp.bfloat16)

    wpw = jnp.zeros((D, D), f32).at[:Cin].set(pw_w)

    c = jnp.float32(_C)
    w12 = jnp.zeros((2 * D, 2 * D), f32)
    w12 = w12.at[:D, :D].set(up_w1[1])
    w12 = w12.at[:D, D:].set(up_w1[2])
    w12 = w12.at[D:, D:].set(up_w1[0])
    w12 = w12 * c
    b1 = up_b1 * c

    w2cat = jnp.concatenate([up_w2[k] for k in range(5)], axis=-1) * (c * c)
    b2 = up_b2 * c
    upg = up_ln_g * c
    upb = up_ln_b * c

    ds, up = pl.pallas_call(
        _backbone_kernel,
        out_shape=(
            jax.ShapeDtypeStruct((B, Tp, D), f32),
            jax.ShapeDtypeStruct((B, 10 * Tp, D), f32),
        ),
        grid=(B // 4,),
        in_specs=[
            pl.BlockSpec((4, Tp, stride * Cin), lambda b: (b, 0, 0)),
            pl.BlockSpec((stride * Cin, 3 * D), lambda b: (0, 0)),
            pl.BlockSpec((D, D), lambda b: (0, 0)),
            pl.BlockSpec((1, D), lambda b: (0, 0)),
            pl.BlockSpec((1, D), lambda b: (0, 0)),
            pl.BlockSpec((2 * D, 2 * D), lambda b: (0, 0)),
            pl.BlockSpec((1, D), lambda b: (0, 0)),
            pl.BlockSpec((D, 5 * D), lambda b: (0, 0)),
            pl.BlockSpec((1, D), lambda b: (0, 0)),
            pl.BlockSpec((1, D), lambda b: (0, 0)),
            pl.BlockSpec((1, D), lambda b: (0, 0)),
        ],
        out_specs=(
            pl.BlockSpec((4, Tp, D), lambda b: (b, 0, 0)),
            pl.BlockSpec((4, 10 * Tp, D), lambda b: (b, 0, 0)),
        ),
        compiler_params=pltpu.CompilerParams(
            dimension_semantics=("parallel",)),
    )(xr, wall, wpw, ds_ln_g, ds_ln_b, w12, b1, w2cat, b2, upg, upb)

    return ds, up


# speedup vs baseline: 1.0009x; 1.0009x over previous
import jax
import jax.numpy as jnp
from jax.experimental import pallas as pl
from jax.experimental.pallas import tpu as pltpu

_C = 0.7071067811865476


def _backbone_kernel(xr_ref, wall_ref, wpw_ref, dsg_ref, dsb_ref,
                     w12_ref, b1_ref, w2_ref, b2_ref, upg_ref, upb_ref,
                     ds_ref, up_ref):
    D = dsg_ref.shape[-1]
    nb = xr_ref.shape[0]
    Tp = xr_ref.shape[1]
    z1 = jnp.zeros((1, D), jnp.float32)
    b1 = b1_ref[...]
    b2 = b2_ref[...]
    gam = upg_ref[...]
    bet = upb_ref[...]
    for i in range(nb):
        X = xr_ref[i]
        A = jnp.dot(X, wall_ref[...], preferred_element_type=jnp.float32)
        acc = (A[:, :D]
               + jnp.concatenate([z1, A[:-1, D:2 * D]], axis=0)
               + jnp.concatenate([A[1:, 2 * D:], z1], axis=0))
        dw = jnp.maximum(acc, 0.0)
        pw = jnp.dot(dw, wpw_ref[...], preferred_element_type=jnp.float32)
        pw = jnp.maximum(pw, 0.0)
        mu = jnp.mean(pw, axis=-1, keepdims=True)
        var = jnp.mean(jnp.square(pw - mu), axis=-1, keepdims=True)
        ds = ((pw - mu) * jax.lax.rsqrt(var + 1e-5) * dsg_ref[...]
              + dsb_ref[...])
        ds_ref[i] = ds

        xn = jnp.concatenate([ds[1:], z1], axis=0)
        X2 = jnp.concatenate([ds, xn], axis=1)
        A2 = jnp.dot(X2, w12_ref[...], preferred_element_type=jnp.float32)
        ge = A2[:, :D] + b1
        go = A2[:, D:] + b1
        h_even = ge * (1.0 + jax.lax.erf(ge))
        h_odd = go * (1.0 + jax.lax.erf(go))

        for j, h in enumerate((h_even, h_odd)):
            Y = jnp.dot(h, w2_ref[...], preferred_element_type=jnp.float32)
            for k in range(5):
                y = Y[:, k * D:(k + 1) * D] + b2
                zz = y * (1.0 + jax.lax.erf(y))
                m2 = jnp.mean(zz, axis=-1, keepdims=True)
                v2 = jnp.mean(jnp.square(zz - m2), axis=-1, keepdims=True)
                z = (zz - m2) * jax.lax.rsqrt(v2 + 2e-5) * gam + bet
                out = (_C * z) * (1.0 + jax.lax.erf(z))
                up_ref[i, (5 * j + k)::10, :] = out


def kernel(x, x_len, dw_w, pw_w, ds_ln_g, ds_ln_b, up_w1, up_b1, up_w2, up_b2,
           up_ln_g, up_ln_b):
    del x_len
    B, T, Cin = x.shape
    D = pw_w.shape[1]
    stride = 10
    Tp = T // stride

    f32 = jnp.float32
    xr = x.astype(jnp.bfloat16).reshape(B, Tp, stride * Cin)

    E = jnp.eye(Cin, dtype=f32)
    blk_c = dw_w[3:13, 0, :, None] * E[None]
    blk_p = jnp.zeros((stride, Cin, Cin), f32).at[7:10].set(
        dw_w[0:3, 0, :, None] * E[None])
    blk_n = jnp.zeros((stride, Cin, Cin), f32).at[0:2].set(
        dw_w[13:15, 0, :, None] * E[None])
    wall = jnp.zeros((stride * Cin, 3 * D), f32)
    wall = wall.at[:, 0:Cin].set(blk_c.reshape(stride * Cin, Cin))
    wall = wall.at[:, D:D + Cin].set(blk_p.reshape(stride * Cin, Cin))
    wall = wall.at[:, 2 * D:2 * D + Cin].set(blk_n.reshape(stride * Cin, Cin))
    wall = wall.astype(jnp.bfloat16)

    wpw = jnp.zeros((D, D), f32).at[:Cin].set(pw_w)

    c = jnp.float32(_C)
    w12 = jnp.zeros((2 * D, 2 * D), f32)
    w12 = w12.at[:D, :D].set(up_w1[1])
    w12 = w12.at[:D, D:].set(up_w1[2])
    w12 = w12.at[D:, D:].set(up_w1[0])
    w12 = w12 * c
    b1 = up_b1 * c

    w2cat = jnp.concatenate([up_w2[k] for k in range(5)], axis=-1) * (c * c)
    b2 = up_b2 * c
    upg = up_ln_g * c
    upb = up_ln_b * c

    ds, up = pl.pallas_call(
        _backbone_kernel,
        out_shape=(
            jax.ShapeDtypeStruct((B, Tp, D), f32),
            jax.ShapeDtypeStruct((B, 10 * Tp, D), f32),
        ),
        grid=(B // 2,),
        in_specs=[
            pl.BlockSpec((2, Tp, stride * Cin), lambda b: (b, 0, 0)),
            pl.BlockSpec((stride * Cin, 3 * D), lambda b: (0, 0)),
            pl.BlockSpec((D, D), lambda b: (0, 0)),
            pl.BlockSpec((1, D), lambda b: (0, 0)),
            pl.BlockSpec((1, D), lambda b: (0, 0)),
            pl.BlockSpec((2 * D, 2 * D), lambda b: (0, 0)),
            pl.BlockSpec((1, D), lambda b: (0, 0)),
            pl.BlockSpec((D, 5 * D), lambda b: (0, 0)),
            pl.BlockSpec((1, D), lambda b: (0, 0)),
            pl.BlockSpec((1, D), lambda b: (0, 0)),
            pl.BlockSpec((1, D), lambda b: (0, 0)),
        ],
        out_specs=(
            pl.BlockSpec((2, Tp, D), lambda b: (b, 0, 0)),
            pl.BlockSpec((2, 10 * Tp, D), lambda b: (b, 0, 0)),
        ),
        compiler_params=pltpu.CompilerParams(
            dimension_semantics=("parallel",)),
    )(xr, wall, wpw, ds_ln_g, ds_ln_b, w12, b1, w2cat, b2, upg, upb)

    return ds, up


# speedup vs baseline: 1.0104x; 1.0095x over previous
import jax
import jax.numpy as jnp
from jax.experimental import pallas as pl
from jax.experimental.pallas import tpu as pltpu

_C = 0.7071067811865476


def _backbone_kernel(xr_ref, wall_ref, wpw_ref, dsg_ref, dsb_ref,
                     w12_ref, b1_ref, w2_ref, b2_ref, upg_ref, upb_ref,
                     ds_ref, up_ref):
    D = dsg_ref.shape[-1]
    nb = xr_ref.shape[0]
    Tp = xr_ref.shape[1]
    z1 = jnp.zeros((1, D), jnp.float32)
    b1 = b1_ref[...]
    b2 = b2_ref[...]
    gam = upg_ref[...]
    bet = upb_ref[...]
    for i in range(nb):
        X = xr_ref[i]
        A = jnp.dot(X, wall_ref[...], preferred_element_type=jnp.float32)
        acc = (A[:, :D]
               + jnp.concatenate([z1, A[:-1, D:2 * D]], axis=0)
               + jnp.concatenate([A[1:, 2 * D:], z1], axis=0))
        dw = jnp.maximum(acc, 0.0)
        pw = jnp.dot(dw, wpw_ref[...], preferred_element_type=jnp.float32)
        pw = jnp.maximum(pw, 0.0)
        mu = jnp.mean(pw, axis=-1, keepdims=True)
        var = jnp.mean(jnp.square(pw - mu), axis=-1, keepdims=True)
        ds = ((pw - mu) * jax.lax.rsqrt(var + 1e-5) * dsg_ref[...]
              + dsb_ref[...])
        ds_ref[i] = ds

        A2 = jnp.dot(ds, w12_ref[...], preferred_element_type=jnp.float32)
        ge = A2[:, :D] + b1
        go = (A2[:, D:2 * D]
              + jnp.concatenate([A2[1:, 2 * D:], z1], axis=0) + b1)
        h_even = ge * (1.0 + jax.lax.erf(ge))
        h_odd = go * (1.0 + jax.lax.erf(go))

        for j, h in enumerate((h_even, h_odd)):
            Y = jnp.dot(h, w2_ref[...], preferred_element_type=jnp.float32)
            for k in range(5):
                y = Y[:, k * D:(k + 1) * D] + b2
                zz = y * (1.0 + jax.lax.erf(y))
                m2 = jnp.mean(zz, axis=-1, keepdims=True)
                v2 = jnp.mean(jnp.square(zz - m2), axis=-1, keepdims=True)
                z = (zz - m2) * jax.lax.rsqrt(v2 + 2e-5) * gam + bet
                out = (_C * z) * (1.0 + jax.lax.erf(z))
                up_ref[i, (5 * j + k)::10, :] = out


def kernel(x, x_len, dw_w, pw_w, ds_ln_g, ds_ln_b, up_w1, up_b1, up_w2, up_b2,
           up_ln_g, up_ln_b):
    del x_len
    B, T, Cin = x.shape
    D = pw_w.shape[1]
    stride = 10
    Tp = T // stride

    f32 = jnp.float32
    xr = x.astype(jnp.bfloat16).reshape(B, Tp, stride * Cin)

    E = jnp.eye(Cin, dtype=f32)
    blk_c = dw_w[3:13, 0, :, None] * E[None]
    blk_p = jnp.zeros((stride, Cin, Cin), f32).at[7:10].set(
        dw_w[0:3, 0, :, None] * E[None])
    blk_n = jnp.zeros((stride, Cin, Cin), f32).at[0:2].set(
        dw_w[13:15, 0, :, None] * E[None])
    wall = jnp.zeros((stride * Cin, 3 * D), f32)
    wall = wall.at[:, 0:Cin].set(blk_c.reshape(stride * Cin, Cin))
    wall = wall.at[:, D:D + Cin].set(blk_p.reshape(stride * Cin, Cin))
    wall = wall.at[:, 2 * D:2 * D + Cin].set(blk_n.reshape(stride * Cin, Cin))
    wall = wall.astype(jnp.bfloat16)

    wpw = jnp.zeros((D, D), f32).at[:Cin].set(pw_w)

    c = jnp.float32(_C)
    w12 = jnp.concatenate([up_w1[1], up_w1[2], up_w1[0]], axis=-1) * c
    b1 = up_b1 * c

    w2cat = jnp.concatenate([up_w2[k] for k in range(5)], axis=-1) * (c * c)
    b2 = up_b2 * c
    upg = up_ln_g * c
    upb = up_ln_b * c

    ds, up = pl.pallas_call(
        _backbone_kernel,
        out_shape=(
            jax.ShapeDtypeStruct((B, Tp, D), f32),
            jax.ShapeDtypeStruct((B, 10 * Tp, D), f32),
        ),
        grid=(B // 2,),
        in_specs=[
            pl.BlockSpec((2, Tp, stride * Cin), lambda b: (b, 0, 0)),
            pl.BlockSpec((stride * Cin, 3 * D), lambda b: (0, 0)),
            pl.BlockSpec((D, D), lambda b: (0, 0)),
            pl.BlockSpec((1, D), lambda b: (0, 0)),
            pl.BlockSpec((1, D), lambda b: (0, 0)),
            pl.BlockSpec((D, 3 * D), lambda b: (0, 0)),
            pl.BlockSpec((1, D), lambda b: (0, 0)),
            pl.BlockSpec((D, 5 * D), lambda b: (0, 0)),
            pl.BlockSpec((1, D), lambda b: (0, 0)),
            pl.BlockSpec((1, D), lambda b: (0, 0)),
            pl.BlockSpec((1, D), lambda b: (0, 0)),
        ],
        out_specs=(
            pl.BlockSpec((2, Tp, D), lambda b: (b, 0, 0)),
            pl.BlockSpec((2, 10 * Tp, D), lambda b: (b, 0, 0)),
        ),
        compiler_params=pltpu.CompilerParams(
            dimension_semantics=("parallel",)),
    )(xr, wall, wpw, ds_ln_g, ds_ln_b, w12, b1, w2cat, b2, upg, upb)

    return ds, up
